# baseline (speedup 1.0000x reference)
import functools
from itertools import combinations

import numpy as np
import jax
import jax.numpy as jnp
from jax import lax
from jax.experimental import pallas as pl
from jax.experimental.pallas import tpu as pltpu
from jax.experimental.pallas import tpu_sc as plsc

_N_WIRES = 16
_N_SHOTS = 1024.0
_N_STATES = 1 << _N_WIRES
_IDX = [(1 << (_N_WIRES - 1 - a)) + (1 << (_N_WIRES - 1 - b))
        for a, b in combinations(range(_N_WIRES), 2)]
_NB = len(_IDX)


_TC_BR = 32
_SC_ROWS = 128
_TC_ROWS = 512 - _SC_ROWS


def _tc_body(x_ref, o_ref):
    x = x_ref[...]
    s = jnp.sum(x, axis=1)
    scale = _N_SHOTS / s
    cols = [x[:, c] for c in _IDX]
    g = jnp.stack(cols, axis=1)
    o_ref[...] = g * scale[:, None]


def _tc_call(state):
    return pl.pallas_call(
        _tc_body,
        grid=(_TC_ROWS // _TC_BR,),
        in_specs=[pl.BlockSpec((_TC_BR, _N_STATES), lambda i: (i, 0))],
        out_specs=pl.BlockSpec((_TC_BR, _NB), lambda i: (i, 0)),
        out_shape=jax.ShapeDtypeStruct((_TC_ROWS, _NB), jnp.float32),
    )(state)


_CW = _N_STATES // 2
_NW = 32
_RPW = _SC_ROWS // _NW


def _make_slot_tables():
    lower = [(j, s) for j, s in enumerate(_IDX) if s < _CW]
    upper = [(j, s) for j, s in enumerate(_IDX) if s >= _CW]
    src = np.zeros(128, np.int32)
    dst = np.zeros(128, np.int32)
    for slot, (j, s) in enumerate(lower):
        src[slot], dst[slot] = s, j
    for p, slot in enumerate(range(len(lower), 112)):
        src[slot], dst[slot] = 0, 121 + p
    for slot, (j, s) in enumerate(upper, start=112):
        src[slot], dst[slot] = s - _CW, j
    src[127], dst[127] = 0, 127
    return src, dst


_SRC_TAB, _DST_TAB = _make_slot_tables()
_CHUNK_OF_VEC = [0] * 7 + [1]


_NRING = 3


def _make_sc_call(base_row, n_rows, n_subcores, name):
    rpw = n_rows // n_subcores

    def _sc_body(state_hbm, src_hbm, dst_hbm, out_ref,
                 buf0, buf1, buf2, srcv, dstv, raw, outv, sem0, sem1, sem2):
        wid = lax.axis_index("s")
        base = base_row + wid * rpw
        pltpu.sync_copy(src_hbm, srcv)
        pltpu.sync_copy(dst_hbm, dstv)
        bufs = (buf0, buf1, buf2)
        sems = (sem0, sem1, sem2)
        seq = [(r, c) for r in range(rpw) for c in range(2)]

        def issue(j):
            r, c = seq[j]
            return pltpu.async_copy(
                state_hbm.at[base + r, pl.ds(c * _CW, _CW)],
                bufs[j % _NRING], sems[j % _NRING])

        pend = {0: issue(0), 1: issue(1)}
        acc = (jnp.zeros((16,), jnp.float32),) * 8
        for j, (r, c) in enumerate(seq):
            pend.pop(j).wait()
            if j + 2 < len(seq):
                pend[j + 2] = issue(j + 2)
            buf = bufs[j % _NRING]

            def body(i, a, buf=buf):
                b = [buf[pl.ds(i * 256 + t * 16, 16)] for t in range(16)]
                a = tuple(a[t] + b[t] for t in range(8))
                return tuple(a[t] + b[8 + t] for t in range(8))

            acc = lax.fori_loop(0, _CW // 256, body, acc)
            for k in range(8):
                if _CHUNK_OF_VEC[k] != c:
                    continue
                sv = srcv[pl.ds(k * 16, 16)]
                raw[pl.ds(k * 16, 16)] = plsc.load_gather(buf, [sv])
            if c == 1:
                t01 = (acc[0] + acc[1]) + (acc[2] + acc[3])
                t23 = (acc[4] + acc[5]) + (acc[6] + acc[7])
                total = jnp.sum(t01 + t23)
                scale = jnp.full((16,), _N_SHOTS, jnp.float32) / (
                    jnp.ones((16,), jnp.float32) * total)
                for k in range(8):
                    dv = dstv[pl.ds(k * 16, 16)]
                    plsc.store_scatter(outv, [dv],
                                       raw[pl.ds(k * 16, 16)] * scale)
                pltpu.sync_copy(outv, out_ref.at[base - base_row + r])
                acc = (jnp.zeros((16,), jnp.float32),) * 8

    return functools.partial(
        pl.kernel,
        out_type=jax.ShapeDtypeStruct((n_rows, 128), jnp.float32),
        mesh=plsc.VectorSubcoreMesh(core_axis_name="c", subcore_axis_name="s",
                                    num_cores=1, num_subcores=n_subcores),
        compiler_params=pltpu.CompilerParams(needs_layout_passes=False),
        name=name,
        scratch_types=[
            pltpu.VMEM((_CW,), jnp.float32),
            pltpu.VMEM((_CW,), jnp.float32),
            pltpu.VMEM((_CW,), jnp.float32),
            pltpu.VMEM((128,), jnp.int32),
            pltpu.VMEM((128,), jnp.int32),
            pltpu.VMEM((128,), jnp.float32),
            pltpu.VMEM((128,), jnp.float32),
            pltpu.SemaphoreType.DMA,
            pltpu.SemaphoreType.DMA,
            pltpu.SemaphoreType.DMA,
        ],
    )(_sc_body)


_SC_HALF = _SC_ROWS // 2
_sc_call_a = _make_sc_call(_TC_ROWS, _SC_HALF, 16, "sc_half_a")
_sc_call_b = _make_sc_call(_TC_ROWS + _SC_HALF, _SC_HALF, 16, "sc_half_b")


def kernel(state):
    tc_out = _tc_call(state)
    src = jnp.asarray(_SRC_TAB)
    dst = jnp.asarray(_DST_TAB)
    sc_a = _sc_call_a(state, src, dst)
    sc_b = _sc_call_b(state, src, dst)
    return jnp.concatenate([tc_out, sc_a[:, :_NB], sc_b[:, :_NB]], axis=0)


# speedup vs baseline: 1.1608x; 1.1608x over previous
import functools
from itertools import combinations

import numpy as np
import jax
import jax.numpy as jnp
from jax import lax
from jax.experimental import pallas as pl
from jax.experimental.pallas import tpu as pltpu
from jax.experimental.pallas import tpu_sc as plsc

_N_WIRES = 16
_N_SHOTS = 1024.0
_N_STATES = 1 << _N_WIRES
_IDX = [(1 << (_N_WIRES - 1 - a)) + (1 << (_N_WIRES - 1 - b))
        for a, b in combinations(range(_N_WIRES), 2)]
_NB = len(_IDX)


_TC_BR = 32
_SC_ROWS = 64
_TC_ROWS = 512 - _SC_ROWS


def _tc_body(x_ref, o_ref):
    x = x_ref[...]
    s = jnp.sum(x, axis=1)
    scale = _N_SHOTS / s
    cols = [x[:, c] for c in _IDX]
    g = jnp.stack(cols, axis=1)
    o_ref[...] = g * scale[:, None]


def _tc_call(state):
    return pl.pallas_call(
        _tc_body,
        grid=(_TC_ROWS // _TC_BR,),
        in_specs=[pl.BlockSpec((_TC_BR, _N_STATES), lambda i: (i, 0))],
        out_specs=pl.BlockSpec((_TC_BR, _NB), lambda i: (i, 0)),
        out_shape=jax.ShapeDtypeStruct((_TC_ROWS, _NB), jnp.float32),
    )(state)


_CW = _N_STATES // 2
_NW = 32
_RPW = _SC_ROWS // _NW


def _make_slot_tables():
    lower = [(j, s) for j, s in enumerate(_IDX) if s < _CW]
    upper = [(j, s) for j, s in enumerate(_IDX) if s >= _CW]
    src = np.zeros(128, np.int32)
    dst = np.zeros(128, np.int32)
    for slot, (j, s) in enumerate(lower):
        src[slot], dst[slot] = s, j
    for p, slot in enumerate(range(len(lower), 112)):
        src[slot], dst[slot] = 0, 121 + p
    for slot, (j, s) in enumerate(upper, start=112):
        src[slot], dst[slot] = s - _CW, j
    src[127], dst[127] = 0, 127
    return src, dst


_SRC_TAB, _DST_TAB = _make_slot_tables()
_CHUNK_OF_VEC = [0] * 7 + [1]


_NRING = 3


def _make_sc_call(base_row, n_rows, num_cores, name):
    n_workers = num_cores * 16
    rpw = n_rows // n_workers

    def _sc_body(state_hbm, src_hbm, dst_hbm, out_ref,
                 buf0, buf1, buf2, srcv, dstv, raw, outv, sem0, sem1, sem2):
        if num_cores == 1:
            wid = lax.axis_index("s")
        else:
            wid = lax.axis_index("s") * num_cores + lax.axis_index("c")
        base = base_row + wid * rpw
        pltpu.sync_copy(src_hbm, srcv)
        pltpu.sync_copy(dst_hbm, dstv)
        bufs = (buf0, buf1, buf2)
        sems = (sem0, sem1, sem2)
        seq = [(r, c) for r in range(rpw) for c in range(2)]

        def issue(j):
            r, c = seq[j]
            return pltpu.async_copy(
                state_hbm.at[base + r, pl.ds(c * _CW, _CW)],
                bufs[j % _NRING], sems[j % _NRING])

        pend = {0: issue(0), 1: issue(1)}
        acc = (jnp.zeros((16,), jnp.float32),) * 8
        for j, (r, c) in enumerate(seq):
            pend.pop(j).wait()
            if j + 2 < len(seq):
                pend[j + 2] = issue(j + 2)
            buf = bufs[j % _NRING]

            def body(i, a, buf=buf):
                b = [buf[pl.ds(i * 256 + t * 16, 16)] for t in range(16)]
                a = tuple(a[t] + b[t] for t in range(8))
                return tuple(a[t] + b[8 + t] for t in range(8))

            acc = lax.fori_loop(0, _CW // 256, body, acc)
            for k in range(8):
                if _CHUNK_OF_VEC[k] != c:
                    continue
                sv = srcv[pl.ds(k * 16, 16)]
                raw[pl.ds(k * 16, 16)] = plsc.load_gather(buf, [sv])
            if c == 1:
                t01 = (acc[0] + acc[1]) + (acc[2] + acc[3])
                t23 = (acc[4] + acc[5]) + (acc[6] + acc[7])
                total = jnp.sum(t01 + t23)
                scale = jnp.full((16,), _N_SHOTS, jnp.float32) / (
                    jnp.ones((16,), jnp.float32) * total)
                for k in range(8):
                    dv = dstv[pl.ds(k * 16, 16)]
                    plsc.store_scatter(outv, [dv],
                                       raw[pl.ds(k * 16, 16)] * scale)
                pltpu.sync_copy(outv, out_ref.at[base - base_row + r])
                acc = (jnp.zeros((16,), jnp.float32),) * 8

    return functools.partial(
        pl.kernel,
        out_type=jax.ShapeDtypeStruct((n_rows, 128), jnp.float32),
        mesh=plsc.VectorSubcoreMesh(core_axis_name="c", subcore_axis_name="s",
                                    num_cores=num_cores, num_subcores=16),
        compiler_params=pltpu.CompilerParams(needs_layout_passes=False),
        name=name,
        scratch_types=[
            pltpu.VMEM((_CW,), jnp.float32),
            pltpu.VMEM((_CW,), jnp.float32),
            pltpu.VMEM((_CW,), jnp.float32),
            pltpu.VMEM((128,), jnp.int32),
            pltpu.VMEM((128,), jnp.int32),
            pltpu.VMEM((128,), jnp.float32),
            pltpu.VMEM((128,), jnp.float32),
            pltpu.SemaphoreType.DMA,
            pltpu.SemaphoreType.DMA,
            pltpu.SemaphoreType.DMA,
        ],
    )(_sc_body)


_sc_call = _make_sc_call(_TC_ROWS, _SC_ROWS, 2, "sc_rows")


def kernel(state):
    tc_out = _tc_call(state)
    sc_out = _sc_call(state, jnp.asarray(_SRC_TAB), jnp.asarray(_DST_TAB))
    return jnp.concatenate([tc_out, sc_out[:, :_NB]], axis=0)
